# baseline (speedup 1.0000x reference)
import functools

import jax
import jax.numpy as jnp
from jax import lax
from jax.experimental import pallas as pl
from jax.experimental.pallas import tpu as pltpu
from jax.experimental.pallas import tpu_sc as plsc

EMBEDDING_DIM = 32


def _build_emb(S, Bb, V, D, num_cores, num_subcores):
    nw = num_cores * num_subcores
    bw = Bb // nw
    ntb = bw // 128
    ntd = D // 8
    S_pad = (S + 7) // 8 * 8
    assert S == 50 and (S - 2) % 6 == 0
    n_blocks = (S - 2) // 8
    mesh = plsc.VectorSubcoreMesh(core_axis_name="c", subcore_axis_name="s")

    @functools.partial(
        pl.kernel,
        mesh=mesh,
        out_type=jax.ShapeDtypeStruct((S, ntd, Bb // 128, 8, 128), jnp.float32),
        scratch_types=[
            pltpu.VMEM((8, bw), jnp.int32),
            pltpu.VMEM((4, bw, D), jnp.float32),
            pltpu.VMEM((2, ntd, ntb, 8, 128), jnp.float32),
            [pltpu.SemaphoreType.DMA] * 8,
            [pltpu.SemaphoreType.DMA] * 4,
            [pltpu.SemaphoreType.DMA] * 2,
        ],
        compiler_params=pltpu.CompilerParams(
            use_tc_tiling_on_sc=False, needs_layout_passes=False),
    )
    def emb(xT, w_rows, out5, idx_v, stage, outstage, isems, gsems, osems):
        wid = lax.axis_index("s") * num_cores + lax.axis_index("c")
        b0 = wid * bw
        tb0 = wid * ntb
        iota16 = lax.iota(jnp.int32, 16)
        cvs = [lax.rem(iota16 + d0, 16) + 16 * h
               for h in range(D // 16) for d0 in range(16)]
        cvs = [(cv, lax.shift_right_logical(cv, 3), lax.bitwise_and(cv, 7))
               for cv in cvs]

        def fire_idx(s, ph):
            i = ph % 8
            pltpu.async_copy(xT.at[s, pl.ds(b0, bw)], idx_v.at[i], isems[i])

        def wait_idx(s, ph):
            i = ph % 8
            pltpu.make_async_copy(
                xT.at[s, pl.ds(b0, bw)], idx_v.at[i], isems[i]).wait()

        def fire_gather(ph):
            i, g = ph % 8, ph % 4
            pltpu.async_copy(w_rows.at[idx_v.at[i]], stage.at[g], gsems[g])

        def wait_gather(ph):
            i, g = ph % 8, ph % 4
            pltpu.make_async_copy(
                w_rows.at[idx_v.at[i]], stage.at[g], gsems[g]).wait()

        def fire_out(s, ph):
            o = ph % 2
            for td in range(ntd):
                pltpu.async_copy(
                    outstage.at[o, td], out5.at[s, td, pl.ds(tb0, ntb)],
                    osems[o])

        def drain_out(s_old, ph):
            o = ph % 2
            for td in range(ntd):
                pltpu.make_async_copy(
                    outstage.at[o, td], out5.at[s_old, td, pl.ds(tb0, ntb)],
                    osems[o]).wait()

        def compact(ph):
            g, o = ph % 4, ph % 2

            def body(r, carry):
                rv = iota16 + r * 16
                tbv = lax.shift_right_logical(rv, 7)
                blv = lax.bitwise_and(rv, 127)
                for cv, tdv, dsv in cvs:
                    vals = plsc.load_gather(stage.at[g], [rv, cv])
                    plsc.store_scatter(
                        outstage.at[o], [tdv, tbv, dsv, blv], vals)
                return carry

            lax.fori_loop(0, bw // 16, body, 0)

        for k in range(7):
            fire_idx(jnp.int32(k), k)
        for k in range(4):
            wait_idx(jnp.int32(k), k)
            fire_gather(k)

        def block(b, carry):
            s_base = b * 8
            for j in range(8):
                s = s_base + j
                wait_gather(j)

                @pl.when(s >= 2)
                def _():
                    drain_out(s - 2, j)

                compact(j)

                @pl.when(s + 7 < S)
                def _():
                    fire_idx(s + 7, j + 7)

                @pl.when(s + 4 < S)
                def _():
                    wait_idx(s + 4, j + 4)
                    fire_gather(j + 4)

                fire_out(s, j)
            return carry

        lax.fori_loop(0, n_blocks, block, 0)

        for j, s in ((0, S - 2), (1, S - 1)):
            wait_gather(j)
            drain_out(s - 2, j)
            compact(j)
            fire_out(s, j)
        drain_out(S - 2, 0)
        drain_out(S - 1, 1)

    return emb


def kernel(x, weight):
    Bb, S = x.shape
    V, D = weight.shape
    S_pad = (S + 7) // 8 * 8
    xT = jnp.pad(x.T, ((0, S_pad - S), (0, 0)))
    emb = _build_emb(S, Bb, V, D, num_cores=2, num_subcores=16)
    out5 = emb(xT, weight)
    outT = out5.transpose(0, 1, 3, 2, 4).reshape(S, D, Bb)
    return outT.transpose(2, 0, 1)
